# baseline (speedup 1.0000x reference)
import functools

import jax
import jax.numpy as jnp
from jax import lax
from jax.experimental import pallas as pl
from jax.experimental.pallas import tpu as pltpu
from jax.experimental.pallas import tpu_sc as plsc

B = 2048
FAN1 = 16
FAN2 = 16
EMB = 128
HID = 128
HEADS = 8
HD = HID // HEADS
W32 = EMB // 2

NW = 32
CH = 64


def _sc_gather_all(table, idx2, idx1, idx0):
    n2 = idx2.shape[0] // NW
    n1 = idx1.shape[0] // NW
    w0 = idx0.shape[1]
    g2 = n2 // 4
    assert g2 >= 2 and g2 % 2 == 0 and n1 == 4
    mesh = plsc.VectorSubcoreMesh(core_axis_name="c", subcore_axis_name="s")

    @functools.partial(
        pl.kernel,
        mesh=mesh,
        out_type=[
            jax.ShapeDtypeStruct((idx2.size, EMB), jnp.float32),
            jax.ShapeDtypeStruct((idx1.size, EMB), jnp.float32),
            jax.ShapeDtypeStruct((idx0.size, EMB), jnp.float32),
        ],
        scratch_types=[
            pltpu.VMEM((n2, CH), jnp.int32),
            pltpu.VMEM((n1, CH), jnp.int32),
            pltpu.VMEM((w0,), jnp.int32),
            pltpu.VMEM((8, CH, EMB), jnp.float32),
            pltpu.VMEM((w0, EMB), jnp.float32),
            pltpu.SemaphoreType.DMA,
            pltpu.SemaphoreType.DMA,
            pltpu.SemaphoreType.DMA,
            pltpu.SemaphoreType.DMA,
            pltpu.SemaphoreType.DMA,
        ],
    )
    def k(tab_hbm, idx2_hbm, idx1_hbm, idx0_hbm, out2_hbm, out1_hbm, out0_hbm,
          idx2_v, idx1_v, idx0_v, bufs, rows_s, gs_a, gs_b, ws_a, ws_b, sem_s):
        wid = lax.axis_index("s") * 2 + lax.axis_index("c")

        pltpu.sync_copy(idx2_hbm.at[pl.ds(wid * n2, n2)], idx2_v)
        pltpu.sync_copy(idx1_hbm.at[pl.ds(wid * n1, n1)], idx1_v)
        pltpu.sync_copy(idx0_hbm.at[wid], idx0_v)

        base2 = wid * n2 * CH
        gsem = (gs_a, gs_b)
        wsem = (ws_a, ws_b)

        def g_copy(c, b, ph):
            return pltpu.make_async_copy(
                tab_hbm.at[idx2_v.at[c]], bufs.at[ph * 4 + b], gsem[ph])

        def w_copy(c, b, ph):
            return pltpu.make_async_copy(
                bufs.at[ph * 4 + b], out2_hbm.at[pl.ds(base2 + c * CH, CH)],
                wsem[ph])

        def fire_g(g, ph):
            for b in range(4):
                g_copy(g * 4 + b, b, ph).start()

        def drain_g(g, ph):
            for b in range(4):
                g_copy(g * 4 + b, b, ph).wait()

        def fire_w(g, ph):
            for b in range(4):
                w_copy(g * 4 + b, b, ph).start()

        def drain_w(g, ph):
            for b in range(4):
                w_copy(g * 4 + b, b, ph).wait()

        fire_g(0, 0)
        fire_g(1, 1)

        def body(i, carry):
            del carry
            g0 = i * 2
            drain_g(g0, 0)
            fire_w(g0, 0)
            drain_g(g0 + 1, 1)
            fire_w(g0 + 1, 1)
            drain_w(g0, 0)
            fire_g(g0 + 2, 0)
            drain_w(g0 + 1, 1)
            fire_g(g0 + 3, 1)
            return 0

        lax.fori_loop(0, g2 // 2 - 1, body, 0)

        ge = g2 - 2
        drain_g(ge, 0)
        fire_w(ge, 0)
        drain_g(ge + 1, 1)
        fire_w(ge + 1, 1)
        drain_w(ge, 0)
        drain_w(ge + 1, 1)

        base1 = wid * n1 * CH
        for b in range(4):
            pltpu.make_async_copy(
                tab_hbm.at[idx1_v.at[b]], bufs.at[b], gs_a).start()
        for b in range(4):
            pltpu.make_async_copy(
                tab_hbm.at[idx1_v.at[b]], bufs.at[b], gs_a).wait()
            pltpu.make_async_copy(
                bufs.at[b], out1_hbm.at[pl.ds(base1 + b * CH, CH)],
                ws_a).start()
        for b in range(4):
            pltpu.make_async_copy(
                bufs.at[b], out1_hbm.at[pl.ds(base1 + b * CH, CH)],
                ws_a).wait()

        pltpu.async_copy(tab_hbm.at[idx0_v], rows_s, sem_s).wait()
        pltpu.sync_copy(rows_s, out0_hbm.at[pl.ds(wid * w0, w0)])

    return k(table, idx2, idx1, idx0)


def _gat_block_f(hs, hn_list, wq, wkv, ws, sb2, ex2, dh):
    scale = float(HD) ** (-0.5)
    dn = (((1,), (1,)), ((), ()))
    dc = (((1,), (0,)), ((), ()))
    f32 = jnp.float32
    bf16 = jnp.bfloat16
    hs = hs.astype(bf16)
    q = lax.dot_general(hs, wq, dn, preferred_element_type=f32)
    qr = q * scale
    nf = len(hn_list)
    vs = []
    ps = []
    for f, hn in enumerate(hn_list):
        hn = hn.astype(bf16)
        kv = lax.dot_general(hn, wkv, dc, preferred_element_type=f32)
        ps.append((kv[:, :HID] * qr).astype(bf16))
        vs.append(kv[:, HID:].astype(bf16))
    s_all = None
    for f in range(0, nf, 2):
        p2 = jnp.concatenate([ps[f], ps[f + 1]], axis=1)
        s = lax.dot_general(p2, sb2[f // 2], dc, preferred_element_type=f32)
        s_all = s if s_all is None else s_all + s
    s_all = jnp.clip(s_all, -75.0, 75.0)
    e = jnp.exp(s_all)
    eb = e.astype(bf16)
    den = lax.dot_general(eb, dh, dc, preferred_element_type=f32)
    agg = None
    for f in range(0, nf, 2):
        w2 = lax.dot_general(eb, ex2[f // 2], dc, preferred_element_type=f32)
        t = w2[:, :HID] * vs[f] + w2[:, HID:] * vs[f + 1]
        agg = t if agg is None else agg + t
    agg = agg / den
    return lax.dot_general(hs, ws, dn, preferred_element_type=f32) + agg


SEED_BLK = 32


def _fused_gat_kernel(h0_ref, h1s_ref, h2_ref, wq1_ref, wkv1_ref, ws1_ref,
                      wq2_ref, wkv2_ref, ws2_ref,
                      sb_ref, ex_ref, dh_ref, out_ref):
    n1 = FAN1 * SEED_BLK
    sb2 = [sb_ref[f] for f in range(FAN2 // 2)]
    ex2 = [ex_ref[f] for f in range(FAN2 // 2)]
    dh = dh_ref[...]
    h1s = h1s_ref[...].reshape(n1, EMB)
    hn1 = [h2_ref[f].reshape(n1, EMB) for f in range(FAN2)]
    h1 = _gat_block_f(h1s, hn1, wq1_ref[...], wkv1_ref[...], ws1_ref[...],
                      sb2, ex2, dh)
    h1 = jnp.maximum(h1, 0.0).astype(jnp.bfloat16)
    hn2 = [h1[j * SEED_BLK:(j + 1) * SEED_BLK] for j in range(FAN1)]
    out = _gat_block_f(h0_ref[...], hn2, wq2_ref[...], wkv2_ref[...],
                       ws2_ref[...], sb2, ex2, dh)
    out_ref[...] = jnp.maximum(out, 0.0)


def _tc_fused(h0, h1s, h2, wq1, wkv1, ws1, wq2, wkv2, ws2,
              sbig, exbig, dhm):
    nb = h0.shape[0]
    grid = nb // SEED_BLK
    wspec = pl.BlockSpec((HID, HID), lambda i: (0, 0))
    kvspec = pl.BlockSpec((HID, 2 * HID), lambda i: (0, 0))
    return pl.pallas_call(
        _fused_gat_kernel,
        grid=(grid,),
        in_specs=[
            pl.BlockSpec((SEED_BLK, EMB), lambda i: (i, 0)),
            pl.BlockSpec((FAN1, SEED_BLK, EMB), lambda i: (0, i, 0)),
            pl.BlockSpec((FAN2, FAN1, SEED_BLK, EMB), lambda i: (0, 0, i, 0)),
            wspec, kvspec, wspec, wspec, kvspec, wspec,
            pl.BlockSpec((FAN2 // 2, 2 * HID, HID), lambda i: (0, 0, 0)),
            pl.BlockSpec((FAN2 // 2, HID, 2 * HID), lambda i: (0, 0, 0)),
            wspec,
        ],
        out_specs=pl.BlockSpec((SEED_BLK, HID), lambda i: (i, 0)),
        out_shape=jax.ShapeDtypeStruct((nb, HID), jnp.float32),
    )(h0, h1s, h2, wq1, wkv1, ws1, wq2, wkv2, ws2, sbig, exbig, dhm)


def _indicators():
    f = jnp.arange(FAN2)[:, None, None]
    d = jnp.arange(HID)[None, :, None]
    c = jnp.arange(HID)[None, None, :]
    sbig = (c == 8 * f + d // HD).astype(jnp.bfloat16)
    exbig = jnp.swapaxes(sbig, 1, 2)
    sbig = sbig.reshape(FAN2 // 2, 2 * HID, HID)
    exbig = jnp.concatenate(
        [exbig[0::2], exbig[1::2]], axis=2)
    cc = jnp.arange(HID)[:, None]
    dd = jnp.arange(HID)[None, :]
    dhm = (cc % HEADS == dd // HD).astype(jnp.bfloat16)
    return sbig, exbig, dhm


NGROUPS = 4


def kernel(seeds, nbr1, nbr2, emb, Wq1, Wk1, Wv1, Ws1, Wq2, Wk2, Wv2, Ws2):
    nbr2_t = nbr2.reshape(B, FAN1, FAN2).transpose(2, 1, 0).astype(jnp.int32)
    nbr1_t = nbr1.reshape(B, FAN1).T.astype(jnp.int32)
    seeds = seeds.astype(jnp.int32)
    bf = jnp.bfloat16
    wkv1 = jnp.concatenate([Wk1.T, Wv1.T], axis=1).astype(bf)
    wkv2 = jnp.concatenate([Wk2.T, Wv2.T], axis=1).astype(bf)
    sbig, exbig, dhm = _indicators()
    wq1, ws1 = Wq1.astype(bf), Ws1.astype(bf)
    wq2, ws2 = Wq2.astype(bf), Ws2.astype(bf)
    bg = B // NGROUPS
    outs = []
    for g in range(NGROUPS):
        sl = slice(g * bg, (g + 1) * bg)
        idx2 = nbr2_t[:, :, sl].reshape(-1, CH)
        idx1 = nbr1_t[:, sl].reshape(-1, CH)
        idx0 = seeds[sl].reshape(NW, bg // NW)
        h2, h1s, h0 = _sc_gather_all(emb, idx2, idx1, idx0)
        h2 = h2.reshape(FAN2, FAN1, bg, EMB)
        h1s = h1s.reshape(FAN1, bg, EMB)
        outs.append(_tc_fused(h0, h1s, h2, wq1, wkv1, ws1,
                              wq2, wkv2, ws2, sbig, exbig, dhm))
    return jnp.concatenate(outs, axis=0)


# speedup vs baseline: 1.0052x; 1.0052x over previous
import functools

import jax
import jax.numpy as jnp
from jax import lax
from jax.experimental import pallas as pl
from jax.experimental.pallas import tpu as pltpu
from jax.experimental.pallas import tpu_sc as plsc

B = 2048
FAN1 = 16
FAN2 = 16
EMB = 128
HID = 128
HEADS = 8
HD = HID // HEADS
W32 = EMB // 2

NW = 32
CH = 64


def _sc_gather_all(table, idx2, idx1, idx0, dest2):
    n2 = idx2.shape[0] // NW
    n1 = idx1.shape[0] // NW
    w0 = idx0.shape[1]
    g2 = n2 // 4
    assert g2 >= 2 and g2 % 2 == 0 and n1 == 4
    mesh = plsc.VectorSubcoreMesh(core_axis_name="c", subcore_axis_name="s")

    @functools.partial(
        pl.kernel,
        mesh=mesh,
        out_type=[
            jax.ShapeDtypeStruct((idx2.size, EMB), jnp.float32),
            jax.ShapeDtypeStruct((idx1.size, EMB), jnp.float32),
            jax.ShapeDtypeStruct((idx0.size, EMB), jnp.float32),
        ],
        scratch_types=[
            pltpu.VMEM((n2, CH), jnp.int32),
            pltpu.VMEM((n2, CH), jnp.int32),
            pltpu.VMEM((n1, CH), jnp.int32),
            pltpu.VMEM((w0,), jnp.int32),
            pltpu.VMEM((8, CH, EMB), jnp.float32),
            pltpu.VMEM((w0, EMB), jnp.float32),
            pltpu.SemaphoreType.DMA,
            pltpu.SemaphoreType.DMA,
            pltpu.SemaphoreType.DMA,
            pltpu.SemaphoreType.DMA,
            pltpu.SemaphoreType.DMA,
        ],
    )
    def k(tab_hbm, idx2_hbm, idx1_hbm, idx0_hbm, dest2_hbm,
          out2_hbm, out1_hbm, out0_hbm,
          idx2_v, dest2_v, idx1_v, idx0_v, bufs, rows_s,
          gs_a, gs_b, ws_a, ws_b, sem_s):
        wid = lax.axis_index("s") * 2 + lax.axis_index("c")

        pltpu.sync_copy(idx2_hbm.at[pl.ds(wid * n2, n2)], idx2_v)
        pltpu.sync_copy(dest2_hbm.at[pl.ds(wid * n2, n2)], dest2_v)
        pltpu.sync_copy(idx1_hbm.at[pl.ds(wid * n1, n1)], idx1_v)
        pltpu.sync_copy(idx0_hbm.at[wid], idx0_v)

        gsem = (gs_a, gs_b)
        wsem = (ws_a, ws_b)

        def g_copy(c, b, ph):
            return pltpu.make_async_copy(
                tab_hbm.at[idx2_v.at[c]], bufs.at[ph * 4 + b], gsem[ph])

        def w_copy(c, b, ph):
            return pltpu.make_async_copy(
                bufs.at[ph * 4 + b], out2_hbm.at[dest2_v.at[c]], wsem[ph])

        def fire_g(g, ph):
            for b in range(4):
                g_copy(g * 4 + b, b, ph).start()

        def drain_g(g, ph):
            for b in range(4):
                g_copy(g * 4 + b, b, ph).wait()

        def fire_w(g, ph):
            for b in range(4):
                w_copy(g * 4 + b, b, ph).start()

        def drain_w(g, ph):
            for b in range(4):
                w_copy(g * 4 + b, b, ph).wait()

        fire_g(0, 0)
        fire_g(1, 1)

        def body(i, carry):
            del carry
            g0 = i * 2
            drain_g(g0, 0)
            fire_w(g0, 0)
            drain_g(g0 + 1, 1)
            fire_w(g0 + 1, 1)
            drain_w(g0, 0)
            fire_g(g0 + 2, 0)
            drain_w(g0 + 1, 1)
            fire_g(g0 + 3, 1)
            return 0

        lax.fori_loop(0, g2 // 2 - 1, body, 0)

        ge = g2 - 2
        drain_g(ge, 0)
        fire_w(ge, 0)
        drain_g(ge + 1, 1)
        fire_w(ge + 1, 1)
        drain_w(ge, 0)
        drain_w(ge + 1, 1)

        base1 = wid * n1 * CH
        for b in range(4):
            pltpu.make_async_copy(
                tab_hbm.at[idx1_v.at[b]], bufs.at[b], gs_a).start()
        for b in range(4):
            pltpu.make_async_copy(
                tab_hbm.at[idx1_v.at[b]], bufs.at[b], gs_a).wait()
            pltpu.make_async_copy(
                bufs.at[b], out1_hbm.at[pl.ds(base1 + b * CH, CH)],
                ws_a).start()
        for b in range(4):
            pltpu.make_async_copy(
                bufs.at[b], out1_hbm.at[pl.ds(base1 + b * CH, CH)],
                ws_a).wait()

        pltpu.async_copy(tab_hbm.at[idx0_v], rows_s, sem_s).wait()
        pltpu.sync_copy(rows_s, out0_hbm.at[pl.ds(wid * w0, w0)])

    return k(table, idx2, idx1, idx0, dest2)


def _gat_block_f(hs, hn_list, wq, wkv, ws, sb2, ex2, dh):
    scale = float(HD) ** (-0.5)
    dn = (((1,), (1,)), ((), ()))
    dc = (((1,), (0,)), ((), ()))
    f32 = jnp.float32
    bf16 = jnp.bfloat16
    hs = hs.astype(bf16)
    q = lax.dot_general(hs, wq, dn, preferred_element_type=f32)
    qr = q * scale
    nf = len(hn_list)
    vs = []
    ps = []
    for f, hn in enumerate(hn_list):
        hn = hn.astype(bf16)
        kv = lax.dot_general(hn, wkv, dc, preferred_element_type=f32)
        ps.append((kv[:, :HID] * qr).astype(bf16))
        vs.append(kv[:, HID:].astype(bf16))
    s_all = None
    for f in range(0, nf, 2):
        p2 = jnp.concatenate([ps[f], ps[f + 1]], axis=1)
        s = lax.dot_general(p2, sb2[f // 2], dc, preferred_element_type=f32)
        s_all = s if s_all is None else s_all + s
    s_all = jnp.clip(s_all, -75.0, 75.0)
    e = jnp.exp(s_all)
    eb = e.astype(bf16)
    den = lax.dot_general(eb, dh, dc, preferred_element_type=f32)
    agg = None
    for f in range(0, nf, 2):
        w2 = lax.dot_general(eb, ex2[f // 2], dc, preferred_element_type=f32)
        t = w2[:, :HID] * vs[f] + w2[:, HID:] * vs[f + 1]
        agg = t if agg is None else agg + t
    agg = agg / den
    return lax.dot_general(hs, ws, dn, preferred_element_type=f32) + agg


SEED_BLK = 32


def _fused_gat_kernel(h0_ref, h1s_ref, h2_ref, wq1_ref, wkv1_ref, ws1_ref,
                      wq2_ref, wkv2_ref, ws2_ref,
                      sb_ref, ex_ref, dh_ref, out_ref):
    n1 = FAN1 * SEED_BLK
    sb2 = [sb_ref[f] for f in range(FAN2 // 2)]
    ex2 = [ex_ref[f] for f in range(FAN2 // 2)]
    dh = dh_ref[...]
    h1s = h1s_ref[...].reshape(n1, EMB)
    hn1 = [h2_ref[f].reshape(n1, EMB) for f in range(FAN2)]
    h1 = _gat_block_f(h1s, hn1, wq1_ref[...], wkv1_ref[...], ws1_ref[...],
                      sb2, ex2, dh)
    h1 = jnp.maximum(h1, 0.0).astype(jnp.bfloat16)
    hn2 = [h1[j * SEED_BLK:(j + 1) * SEED_BLK] for j in range(FAN1)]
    out = _gat_block_f(h0_ref[...], hn2, wq2_ref[...], wkv2_ref[...],
                       ws2_ref[...], sb2, ex2, dh)
    out_ref[...] = jnp.maximum(out, 0.0)


def _tc_fused(h0, h1s, h2, wq1, wkv1, ws1, wq2, wkv2, ws2,
              sbig, exbig, dhm):
    nb = h0.shape[0]
    grid = nb // SEED_BLK
    wspec = pl.BlockSpec((HID, HID), lambda i: (0, 0))
    kvspec = pl.BlockSpec((HID, 2 * HID), lambda i: (0, 0))
    return pl.pallas_call(
        _fused_gat_kernel,
        grid=(grid,),
        in_specs=[
            pl.BlockSpec((SEED_BLK, EMB), lambda i: (i, 0)),
            pl.BlockSpec((FAN1, SEED_BLK, EMB), lambda i: (0, i, 0)),
            pl.BlockSpec((FAN2, FAN1, SEED_BLK, EMB), lambda i: (0, 0, i, 0)),
            wspec, kvspec, wspec, wspec, kvspec, wspec,
            pl.BlockSpec((FAN2 // 2, 2 * HID, HID), lambda i: (0, 0, 0)),
            pl.BlockSpec((FAN2 // 2, HID, 2 * HID), lambda i: (0, 0, 0)),
            wspec,
        ],
        out_specs=pl.BlockSpec((SEED_BLK, HID), lambda i: (i, 0)),
        out_shape=jax.ShapeDtypeStruct((nb, HID), jnp.float32),
    )(h0, h1s, h2, wq1, wkv1, ws1, wq2, wkv2, ws2, sbig, exbig, dhm)


def _indicators():
    f = jnp.arange(FAN2)[:, None, None]
    d = jnp.arange(HID)[None, :, None]
    c = jnp.arange(HID)[None, None, :]
    sbig = (c == 8 * f + d // HD).astype(jnp.bfloat16)
    exbig = jnp.swapaxes(sbig, 1, 2)
    sbig = sbig.reshape(FAN2 // 2, 2 * HID, HID)
    exbig = jnp.concatenate(
        [exbig[0::2], exbig[1::2]], axis=2)
    cc = jnp.arange(HID)[:, None]
    dd = jnp.arange(HID)[None, :]
    dhm = (cc % HEADS == dd // HD).astype(jnp.bfloat16)
    return sbig, exbig, dhm


NGROUPS = 4


def kernel(seeds, nbr1, nbr2, emb, Wq1, Wk1, Wv1, Ws1, Wq2, Wk2, Wv2, Ws2):
    bg = B // NGROUPS
    p = jnp.arange(bg * FAN1 * FAN2, dtype=jnp.int32)
    dest2 = ((p % FAN2) * (FAN1 * bg) + ((p // FAN2) % FAN1) * bg
             + p // (FAN1 * FAN2)).reshape(-1, CH)
    nbr2_r = nbr2.reshape(-1).astype(jnp.int32)
    nbr1_t = nbr1.reshape(B, FAN1).T.astype(jnp.int32)
    seeds = seeds.astype(jnp.int32)
    bf = jnp.bfloat16
    wkv1 = jnp.concatenate([Wk1.T, Wv1.T], axis=1).astype(bf)
    wkv2 = jnp.concatenate([Wk2.T, Wv2.T], axis=1).astype(bf)
    sbig, exbig, dhm = _indicators()
    wq1, ws1 = Wq1.astype(bf), Ws1.astype(bf)
    wq2, ws2 = Wq2.astype(bf), Ws2.astype(bf)
    outs = []
    for g in range(NGROUPS):
        sl = slice(g * bg, (g + 1) * bg)
        idx2 = nbr2_r[g * bg * FAN1 * FAN2:(g + 1) * bg * FAN1 * FAN2]
        idx2 = idx2.reshape(-1, CH)
        idx1 = nbr1_t[:, sl].reshape(-1, CH)
        idx0 = seeds[sl].reshape(NW, bg // NW)
        h2, h1s, h0 = _sc_gather_all(emb, idx2, idx1, idx0, dest2)
        h2 = h2.reshape(FAN2, FAN1, bg, EMB)
        h1s = h1s.reshape(FAN1, bg, EMB)
        outs.append(_tc_fused(h0, h1s, h2, wq1, wkv1, ws1,
                              wq2, wkv2, ws2, sbig, exbig, dhm))
    return jnp.concatenate(outs, axis=0)


# speedup vs baseline: 1.0319x; 1.0265x over previous
import functools

import jax
import jax.numpy as jnp
from jax import lax
from jax.experimental import pallas as pl
from jax.experimental.pallas import tpu as pltpu
from jax.experimental.pallas import tpu_sc as plsc

B = 2048
FAN1 = 16
FAN2 = 16
EMB = 128
HID = 128
HEADS = 8
HD = HID // HEADS
W32 = EMB // 2

NW = 32
CH = 64


def _sc_gather_all(table, idx2, idx1, idx0, dest2):
    n2 = idx2.shape[0] // NW
    n1 = idx1.shape[0] // NW
    w0 = idx0.shape[1]
    g2 = n2 // 4
    assert g2 >= 2 and g2 % 2 == 0 and n1 == 4
    mesh = plsc.VectorSubcoreMesh(core_axis_name="c", subcore_axis_name="s")

    @functools.partial(
        pl.kernel,
        mesh=mesh,
        out_type=[
            jax.ShapeDtypeStruct((idx2.size, EMB), jnp.float32),
            jax.ShapeDtypeStruct((idx1.size, EMB), jnp.float32),
            jax.ShapeDtypeStruct((idx0.size, EMB), jnp.float32),
        ],
        scratch_types=[
            pltpu.VMEM((n2, CH), jnp.int32),
            pltpu.VMEM((n2, CH), jnp.int32),
            pltpu.VMEM((n1, CH), jnp.int32),
            pltpu.VMEM((w0,), jnp.int32),
            pltpu.VMEM((8, CH, EMB), jnp.float32),
            pltpu.VMEM((w0, EMB), jnp.float32),
            pltpu.SemaphoreType.DMA,
            pltpu.SemaphoreType.DMA,
            pltpu.SemaphoreType.DMA,
            pltpu.SemaphoreType.DMA,
            pltpu.SemaphoreType.DMA,
        ],
    )
    def k(tab_hbm, idx2_hbm, idx1_hbm, idx0_hbm, dest2_hbm,
          out2_hbm, out1_hbm, out0_hbm,
          idx2_v, dest2_v, idx1_v, idx0_v, bufs, rows_s,
          gs_a, gs_b, ws_a, ws_b, sem_s):
        wid = lax.axis_index("s") * 2 + lax.axis_index("c")

        pltpu.sync_copy(idx2_hbm.at[pl.ds(wid * n2, n2)], idx2_v)
        pltpu.sync_copy(dest2_hbm.at[pl.ds(wid * n2, n2)], dest2_v)
        pltpu.sync_copy(idx1_hbm.at[pl.ds(wid * n1, n1)], idx1_v)
        pltpu.sync_copy(idx0_hbm.at[wid], idx0_v)

        gsem = (gs_a, gs_b)
        wsem = (ws_a, ws_b)

        def g_copy(c, b, ph):
            return pltpu.make_async_copy(
                tab_hbm.at[idx2_v.at[c]], bufs.at[ph * 4 + b], gsem[ph])

        def w_copy(c, b, ph):
            return pltpu.make_async_copy(
                bufs.at[ph * 4 + b], out2_hbm.at[dest2_v.at[c]], wsem[ph])

        def fire_g(g, ph):
            for b in range(4):
                g_copy(g * 4 + b, b, ph).start()

        def drain_g(g, ph):
            for b in range(4):
                g_copy(g * 4 + b, b, ph).wait()

        def fire_w(g, ph):
            for b in range(4):
                w_copy(g * 4 + b, b, ph).start()

        def drain_w(g, ph):
            for b in range(4):
                w_copy(g * 4 + b, b, ph).wait()

        fire_g(0, 0)
        fire_g(1, 1)

        def body(i, carry):
            del carry
            g0 = i * 2
            drain_g(g0, 0)
            fire_w(g0, 0)
            drain_g(g0 + 1, 1)
            fire_w(g0 + 1, 1)
            drain_w(g0, 0)
            fire_g(g0 + 2, 0)
            drain_w(g0 + 1, 1)
            fire_g(g0 + 3, 1)
            return 0

        lax.fori_loop(0, g2 // 2 - 1, body, 0)

        ge = g2 - 2
        drain_g(ge, 0)
        fire_w(ge, 0)
        drain_g(ge + 1, 1)
        fire_w(ge + 1, 1)
        drain_w(ge, 0)
        drain_w(ge + 1, 1)

        base1 = wid * n1 * CH
        for b in range(4):
            pltpu.make_async_copy(
                tab_hbm.at[idx1_v.at[b]], bufs.at[b], gs_a).start()
        for b in range(4):
            pltpu.make_async_copy(
                tab_hbm.at[idx1_v.at[b]], bufs.at[b], gs_a).wait()
            pltpu.make_async_copy(
                bufs.at[b], out1_hbm.at[pl.ds(base1 + b * CH, CH)],
                ws_a).start()
        for b in range(4):
            pltpu.make_async_copy(
                bufs.at[b], out1_hbm.at[pl.ds(base1 + b * CH, CH)],
                ws_a).wait()

        pltpu.async_copy(tab_hbm.at[idx0_v], rows_s, sem_s).wait()
        pltpu.sync_copy(rows_s, out0_hbm.at[pl.ds(wid * w0, w0)])

    return k(table, idx2, idx1, idx0, dest2)


def _gat_block_f(hs, hn_list, wq, wkv, ws, sb2, ex2, dh):
    scale = float(HD) ** (-0.5)
    dn = (((1,), (1,)), ((), ()))
    dc = (((1,), (0,)), ((), ()))
    f32 = jnp.float32
    bf16 = jnp.bfloat16
    hs = hs.astype(bf16)
    q = lax.dot_general(hs, wq, dn, preferred_element_type=f32)
    qr = (q * scale).astype(bf16)
    nf = len(hn_list)
    vs = []
    ps = []
    for f, hn in enumerate(hn_list):
        hn = hn.astype(bf16)
        kv = lax.dot_general(hn, wkv, dc, preferred_element_type=f32)
        ps.append(kv[:, :HID].astype(bf16) * qr)
        vs.append(kv[:, HID:].astype(bf16))
    s_all = None
    for f in range(nf):
        s = lax.dot_general(ps[f], sb2[f], dc, preferred_element_type=f32)
        s_all = s if s_all is None else s_all + s
    s_all = jnp.clip(s_all, -75.0, 75.0)
    e = jnp.exp(s_all)
    eb = e.astype(bf16)
    den = lax.dot_general(eb, dh, dc, preferred_element_type=f32)
    agg = None
    for f in range(0, nf, 2):
        w2 = lax.dot_general(eb, ex2[f // 2], dc, preferred_element_type=f32)
        t = w2[:, :HID] * vs[f] + w2[:, HID:] * vs[f + 1]
        agg = t if agg is None else agg + t
    agg = agg / den
    return lax.dot_general(hs, ws, dn, preferred_element_type=f32) + agg


SEED_BLK = 64


def _fused_gat_kernel(h0_ref, h1s_ref, h2_ref, wq1_ref, wkv1_ref, ws1_ref,
                      wq2_ref, wkv2_ref, ws2_ref,
                      sb_ref, ex_ref, dh_ref, out_ref):
    n1 = FAN1 * SEED_BLK
    sb2 = [sb_ref[f] for f in range(FAN2)]
    ex2 = [ex_ref[f] for f in range(FAN2 // 2)]
    dh = dh_ref[...]
    h1s = h1s_ref[...].reshape(n1, EMB)
    hn1 = [h2_ref[f].reshape(n1, EMB) for f in range(FAN2)]
    h1 = _gat_block_f(h1s, hn1, wq1_ref[...], wkv1_ref[...], ws1_ref[...],
                      sb2, ex2, dh)
    h1 = jnp.maximum(h1, 0.0).astype(jnp.bfloat16)
    hn2 = [h1[j * SEED_BLK:(j + 1) * SEED_BLK] for j in range(FAN1)]
    out = _gat_block_f(h0_ref[...], hn2, wq2_ref[...], wkv2_ref[...],
                       ws2_ref[...], sb2, ex2, dh)
    out_ref[...] = jnp.maximum(out, 0.0)


def _tc_fused(h0, h1s, h2, wq1, wkv1, ws1, wq2, wkv2, ws2,
              sbig, exbig, dhm):
    nb = h0.shape[0]
    grid = nb // SEED_BLK
    wspec = pl.BlockSpec((HID, HID), lambda i: (0, 0))
    kvspec = pl.BlockSpec((HID, 2 * HID), lambda i: (0, 0))
    return pl.pallas_call(
        _fused_gat_kernel,
        grid=(grid,),
        in_specs=[
            pl.BlockSpec((SEED_BLK, EMB), lambda i: (i, 0)),
            pl.BlockSpec((FAN1, SEED_BLK, EMB), lambda i: (0, i, 0)),
            pl.BlockSpec((FAN2, FAN1, SEED_BLK, EMB), lambda i: (0, 0, i, 0)),
            wspec, kvspec, wspec, wspec, kvspec, wspec,
            pl.BlockSpec((FAN2, HID, HID), lambda i: (0, 0, 0)),
            pl.BlockSpec((FAN2 // 2, HID, 2 * HID), lambda i: (0, 0, 0)),
            wspec,
        ],
        out_specs=pl.BlockSpec((SEED_BLK, HID), lambda i: (i, 0)),
        out_shape=jax.ShapeDtypeStruct((nb, HID), jnp.float32),
    )(h0, h1s, h2, wq1, wkv1, ws1, wq2, wkv2, ws2, sbig, exbig, dhm)


def _indicators():
    f = jnp.arange(FAN2)[:, None, None]
    d = jnp.arange(HID)[None, :, None]
    c = jnp.arange(HID)[None, None, :]
    sbig = (c == 8 * f + d // HD).astype(jnp.bfloat16)
    exbig = jnp.swapaxes(sbig, 1, 2)
    exbig = jnp.concatenate(
        [exbig[0::2], exbig[1::2]], axis=2)
    cc = jnp.arange(HID)[:, None]
    dd = jnp.arange(HID)[None, :]
    dhm = (cc % HEADS == dd // HD).astype(jnp.bfloat16)
    return sbig, exbig, dhm


NGROUPS = 4


def kernel(seeds, nbr1, nbr2, emb, Wq1, Wk1, Wv1, Ws1, Wq2, Wk2, Wv2, Ws2):
    bg = B // NGROUPS
    p = jnp.arange(bg * FAN1 * FAN2, dtype=jnp.int32)
    dest2 = ((p % FAN2) * (FAN1 * bg) + ((p // FAN2) % FAN1) * bg
             + p // (FAN1 * FAN2)).reshape(-1, CH)
    nbr2_r = nbr2.reshape(-1).astype(jnp.int32)
    nbr1_t = nbr1.reshape(B, FAN1).T.astype(jnp.int32)
    seeds = seeds.astype(jnp.int32)
    bf = jnp.bfloat16
    wkv1 = jnp.concatenate([Wk1.T, Wv1.T], axis=1).astype(bf)
    wkv2 = jnp.concatenate([Wk2.T, Wv2.T], axis=1).astype(bf)
    sbig, exbig, dhm = _indicators()
    wq1, ws1 = Wq1.astype(bf), Ws1.astype(bf)
    wq2, ws2 = Wq2.astype(bf), Ws2.astype(bf)
    outs = []
    for g in range(NGROUPS):
        sl = slice(g * bg, (g + 1) * bg)
        idx2 = nbr2_r[g * bg * FAN1 * FAN2:(g + 1) * bg * FAN1 * FAN2]
        idx2 = idx2.reshape(-1, CH)
        idx1 = nbr1_t[:, sl].reshape(-1, CH)
        idx0 = seeds[sl].reshape(NW, bg // NW)
        h2, h1s, h0 = _sc_gather_all(emb, idx2, idx1, idx0, dest2)
        h2 = h2.reshape(FAN2, FAN1, bg, EMB)
        h1s = h1s.reshape(FAN1, bg, EMB)
        outs.append(_tc_fused(h0, h1s, h2, wq1, wkv1, ws1,
                              wq2, wkv2, ws2, sbig, exbig, dhm))
    return jnp.concatenate(outs, axis=0)


# speedup vs baseline: 1.0378x; 1.0057x over previous
import functools

import jax
import jax.numpy as jnp
from jax import lax
from jax.experimental import pallas as pl
from jax.experimental.pallas import tpu as pltpu
from jax.experimental.pallas import tpu_sc as plsc

B = 2048
FAN1 = 16
FAN2 = 16
EMB = 128
HID = 128
HEADS = 8
HD = HID // HEADS
W32 = EMB // 2

NW = 32
CH = 64


def _sc_gather_all(table, idx2, idx1, idx0, dest2):
    n2 = idx2.shape[0] // NW
    n1 = idx1.shape[0] // NW
    w0 = idx0.shape[1]
    g2 = n2 // 4
    assert g2 >= 2 and g2 % 2 == 0 and n1 <= 4
    mesh = plsc.VectorSubcoreMesh(core_axis_name="c", subcore_axis_name="s")

    @functools.partial(
        pl.kernel,
        mesh=mesh,
        out_type=[
            jax.ShapeDtypeStruct((idx2.size, EMB), jnp.float32),
            jax.ShapeDtypeStruct((idx1.size, EMB), jnp.float32),
            jax.ShapeDtypeStruct((idx0.size, EMB), jnp.float32),
        ],
        scratch_types=[
            pltpu.VMEM((n2, CH), jnp.int32),
            pltpu.VMEM((n2, CH), jnp.int32),
            pltpu.VMEM((n1, CH), jnp.int32),
            pltpu.VMEM((w0,), jnp.int32),
            pltpu.VMEM((8, CH, EMB), jnp.float32),
            pltpu.VMEM((n1, CH, EMB), jnp.float32),
            pltpu.VMEM((w0, EMB), jnp.float32),
            pltpu.SemaphoreType.DMA,
            pltpu.SemaphoreType.DMA,
            pltpu.SemaphoreType.DMA,
            pltpu.SemaphoreType.DMA,
            pltpu.SemaphoreType.DMA,
            pltpu.SemaphoreType.DMA,
            pltpu.SemaphoreType.DMA,
        ],
    )
    def k(tab_hbm, idx2_hbm, idx1_hbm, idx0_hbm, dest2_hbm,
          out2_hbm, out1_hbm, out0_hbm,
          idx2_v, dest2_v, idx1_v, idx0_v, bufs, bufs1, rows_s,
          gs_a, gs_b, ws_a, ws_b, gs_1, ws_1, sem_s):
        wid = lax.axis_index("s") * 2 + lax.axis_index("c")

        pltpu.sync_copy(idx2_hbm.at[pl.ds(wid * n2, n2)], idx2_v)
        pltpu.sync_copy(dest2_hbm.at[pl.ds(wid * n2, n2)], dest2_v)
        pltpu.sync_copy(idx1_hbm.at[pl.ds(wid * n1, n1)], idx1_v)
        pltpu.sync_copy(idx0_hbm.at[wid], idx0_v)

        base1 = wid * n1 * CH
        for b in range(n1):
            pltpu.make_async_copy(
                tab_hbm.at[idx1_v.at[b]], bufs1.at[b], gs_1).start()
        seed_cp = pltpu.async_copy(tab_hbm.at[idx0_v], rows_s, sem_s)

        gsem = (gs_a, gs_b)
        wsem = (ws_a, ws_b)

        def g_copy(c, b, ph):
            return pltpu.make_async_copy(
                tab_hbm.at[idx2_v.at[c]], bufs.at[ph * 4 + b], gsem[ph])

        def w_copy(c, b, ph):
            return pltpu.make_async_copy(
                bufs.at[ph * 4 + b], out2_hbm.at[dest2_v.at[c]], wsem[ph])

        def fire_g(g, ph):
            for b in range(4):
                g_copy(g * 4 + b, b, ph).start()

        def drain_g(g, ph):
            for b in range(4):
                g_copy(g * 4 + b, b, ph).wait()

        def fire_w(g, ph):
            for b in range(4):
                w_copy(g * 4 + b, b, ph).start()

        def drain_w(g, ph):
            for b in range(4):
                w_copy(g * 4 + b, b, ph).wait()

        fire_g(0, 0)
        fire_g(1, 1)

        def body(i, carry):
            del carry
            g0 = i * 2
            drain_g(g0, 0)
            fire_w(g0, 0)
            drain_g(g0 + 1, 1)
            fire_w(g0 + 1, 1)
            drain_w(g0, 0)
            fire_g(g0 + 2, 0)
            drain_w(g0 + 1, 1)
            fire_g(g0 + 3, 1)
            return 0

        lax.fori_loop(0, g2 // 2 - 1, body, 0)

        ge = g2 - 2
        drain_g(ge, 0)
        fire_w(ge, 0)
        drain_g(ge + 1, 1)
        fire_w(ge + 1, 1)
        drain_w(ge, 0)
        drain_w(ge + 1, 1)

        for b in range(n1):
            pltpu.make_async_copy(
                tab_hbm.at[idx1_v.at[b]], bufs1.at[b], gs_1).wait()
            pltpu.make_async_copy(
                bufs1.at[b], out1_hbm.at[pl.ds(base1 + b * CH, CH)],
                ws_1).start()
        seed_cp.wait()
        pltpu.sync_copy(rows_s, out0_hbm.at[pl.ds(wid * w0, w0)])
        for b in range(n1):
            pltpu.make_async_copy(
                bufs1.at[b], out1_hbm.at[pl.ds(base1 + b * CH, CH)],
                ws_1).wait()

    return k(table, idx2, idx1, idx0, dest2)


def _gat_block_f(hs, hn_list, wq, wkv, ws, sb2, ex2, dh):
    scale = float(HD) ** (-0.5)
    dn = (((1,), (1,)), ((), ()))
    dc = (((1,), (0,)), ((), ()))
    f32 = jnp.float32
    bf16 = jnp.bfloat16
    hs = hs.astype(bf16)
    q = lax.dot_general(hs, wq, dn, preferred_element_type=f32)
    qr = (q * scale).astype(bf16)
    nf = len(hn_list)
    vs = []
    ps = []
    for f, hn in enumerate(hn_list):
        hn = hn.astype(bf16)
        kv = lax.dot_general(hn, wkv, dc, preferred_element_type=f32)
        ps.append(kv[:, :HID].astype(bf16) * qr)
        vs.append(kv[:, HID:].astype(bf16))
    s_all = None
    for f in range(nf):
        s = lax.dot_general(ps[f], sb2[f], dc, preferred_element_type=f32)
        s_all = s if s_all is None else s_all + s
    s_all = jnp.clip(s_all, -75.0, 75.0)
    e = jnp.exp(s_all)
    eb = e.astype(bf16)
    den = lax.dot_general(eb, dh, dc, preferred_element_type=f32)
    agg = None
    for f in range(0, nf, 2):
        w2 = lax.dot_general(eb, ex2[f // 2], dc, preferred_element_type=f32)
        t = w2[:, :HID] * vs[f] + w2[:, HID:] * vs[f + 1]
        agg = t if agg is None else agg + t
    agg = agg / den
    return lax.dot_general(hs, ws, dn, preferred_element_type=f32) + agg


SEED_BLK = 64


def _fused_gat_kernel(h0_ref, h1s_ref, h2_ref, wq1_ref, wkv1_ref, ws1_ref,
                      wq2_ref, wkv2_ref, ws2_ref,
                      sb_ref, ex_ref, dh_ref, out_ref):
    n1 = FAN1 * SEED_BLK
    sb2 = [sb_ref[f] for f in range(FAN2)]
    ex2 = [ex_ref[f] for f in range(FAN2 // 2)]
    dh = dh_ref[...]
    h1s = h1s_ref[...].reshape(n1, EMB)
    hn1 = [h2_ref[f].reshape(n1, EMB) for f in range(FAN2)]
    h1 = _gat_block_f(h1s, hn1, wq1_ref[...], wkv1_ref[...], ws1_ref[...],
                      sb2, ex2, dh)
    h1 = jnp.maximum(h1, 0.0).astype(jnp.bfloat16)
    hn2 = [h1[j * SEED_BLK:(j + 1) * SEED_BLK] for j in range(FAN1)]
    out = _gat_block_f(h0_ref[...], hn2, wq2_ref[...], wkv2_ref[...],
                       ws2_ref[...], sb2, ex2, dh)
    out_ref[...] = jnp.maximum(out, 0.0)


def _tc_fused(h0, h1s, h2, wq1, wkv1, ws1, wq2, wkv2, ws2,
              sbig, exbig, dhm):
    nb = h0.shape[0]
    grid = nb // SEED_BLK
    wspec = pl.BlockSpec((HID, HID), lambda i: (0, 0))
    kvspec = pl.BlockSpec((HID, 2 * HID), lambda i: (0, 0))
    return pl.pallas_call(
        _fused_gat_kernel,
        grid=(grid,),
        in_specs=[
            pl.BlockSpec((SEED_BLK, EMB), lambda i: (i, 0)),
            pl.BlockSpec((FAN1, SEED_BLK, EMB), lambda i: (0, i, 0)),
            pl.BlockSpec((FAN2, FAN1, SEED_BLK, EMB), lambda i: (0, 0, i, 0)),
            wspec, kvspec, wspec, wspec, kvspec, wspec,
            pl.BlockSpec((FAN2, HID, HID), lambda i: (0, 0, 0)),
            pl.BlockSpec((FAN2 // 2, HID, 2 * HID), lambda i: (0, 0, 0)),
            wspec,
        ],
        out_specs=pl.BlockSpec((SEED_BLK, HID), lambda i: (i, 0)),
        out_shape=jax.ShapeDtypeStruct((nb, HID), jnp.float32),
    )(h0, h1s, h2, wq1, wkv1, ws1, wq2, wkv2, ws2, sbig, exbig, dhm)


def _indicators():
    f = jnp.arange(FAN2)[:, None, None]
    d = jnp.arange(HID)[None, :, None]
    c = jnp.arange(HID)[None, None, :]
    sbig = (c == 8 * f + d // HD).astype(jnp.bfloat16)
    exbig = jnp.swapaxes(sbig, 1, 2)
    exbig = jnp.concatenate(
        [exbig[0::2], exbig[1::2]], axis=2)
    cc = jnp.arange(HID)[:, None]
    dd = jnp.arange(HID)[None, :]
    dhm = (cc % HEADS == dd // HD).astype(jnp.bfloat16)
    return sbig, exbig, dhm


NGROUPS = 4


def kernel(seeds, nbr1, nbr2, emb, Wq1, Wk1, Wv1, Ws1, Wq2, Wk2, Wv2, Ws2):
    bg = B // NGROUPS
    p = jnp.arange(bg * FAN1 * FAN2, dtype=jnp.int32)
    dest2 = ((p % FAN2) * (FAN1 * bg) + ((p // FAN2) % FAN1) * bg
             + p // (FAN1 * FAN2)).reshape(-1, CH)
    nbr2_r = nbr2.reshape(-1).astype(jnp.int32)
    nbr1_t = nbr1.reshape(B, FAN1).T.astype(jnp.int32)
    seeds = seeds.astype(jnp.int32)
    bf = jnp.bfloat16
    wkv1 = jnp.concatenate([Wk1.T, Wv1.T], axis=1).astype(bf)
    wkv2 = jnp.concatenate([Wk2.T, Wv2.T], axis=1).astype(bf)
    sbig, exbig, dhm = _indicators()
    wq1, ws1 = Wq1.astype(bf), Ws1.astype(bf)
    wq2, ws2 = Wq2.astype(bf), Ws2.astype(bf)
    outs = []
    for g in range(NGROUPS):
        sl = slice(g * bg, (g + 1) * bg)
        idx2 = nbr2_r[g * bg * FAN1 * FAN2:(g + 1) * bg * FAN1 * FAN2]
        idx2 = idx2.reshape(-1, CH)
        idx1 = nbr1_t[:, sl].reshape(-1, CH)
        idx0 = seeds[sl].reshape(NW, bg // NW)
        h2, h1s, h0 = _sc_gather_all(emb, idx2, idx1, idx0, dest2)
        h2 = h2.reshape(FAN2, FAN1, bg, EMB)
        h1s = h1s.reshape(FAN1, bg, EMB)
        outs.append(_tc_fused(h0, h1s, h2, wq1, wkv1, ws1,
                              wq2, wkv2, ws2, sbig, exbig, dhm))
    return jnp.concatenate(outs, axis=0)


# speedup vs baseline: 1.0430x; 1.0050x over previous
import functools

import jax
import jax.numpy as jnp
from jax import lax
from jax.experimental import pallas as pl
from jax.experimental.pallas import tpu as pltpu
from jax.experimental.pallas import tpu_sc as plsc

B = 2048
FAN1 = 16
FAN2 = 16
EMB = 128
HID = 128
HEADS = 8
HD = HID // HEADS
W32 = EMB // 2

NW = 32
CH = 64


def _sc_gather_all(table, idx2, idx1, idx0, dest2):
    n2 = idx2.shape[0] // NW
    n1 = idx1.shape[0] // NW
    w0 = idx0.shape[1]
    g2 = n2 // 4
    assert g2 >= 2 and g2 % 2 == 0 and n1 <= 4
    mesh = plsc.VectorSubcoreMesh(core_axis_name="c", subcore_axis_name="s")

    @functools.partial(
        pl.kernel,
        mesh=mesh,
        out_type=[
            jax.ShapeDtypeStruct((idx2.size, EMB), jnp.float32),
            jax.ShapeDtypeStruct((idx1.size, EMB), jnp.float32),
            jax.ShapeDtypeStruct((idx0.size, EMB), jnp.float32),
        ],
        scratch_types=[
            pltpu.VMEM((n2, CH), jnp.int32),
            pltpu.VMEM((n2, CH), jnp.int32),
            pltpu.VMEM((n1, CH), jnp.int32),
            pltpu.VMEM((w0,), jnp.int32),
            pltpu.VMEM((8, CH, EMB), jnp.float32),
            pltpu.VMEM((n1, CH, EMB), jnp.float32),
            pltpu.VMEM((w0, EMB), jnp.float32),
            pltpu.SemaphoreType.DMA,
            pltpu.SemaphoreType.DMA,
            pltpu.SemaphoreType.DMA,
            pltpu.SemaphoreType.DMA,
            pltpu.SemaphoreType.DMA,
            pltpu.SemaphoreType.DMA,
            pltpu.SemaphoreType.DMA,
        ],
    )
    def k(tab_hbm, idx2_hbm, idx1_hbm, idx0_hbm, dest2_hbm,
          out2_hbm, out1_hbm, out0_hbm,
          idx2_v, dest2_v, idx1_v, idx0_v, bufs, bufs1, rows_s,
          gs_a, gs_b, ws_a, ws_b, gs_1, ws_1, sem_s):
        wid = lax.axis_index("s") * 2 + lax.axis_index("c")

        pltpu.sync_copy(idx2_hbm.at[pl.ds(wid * n2, n2)], idx2_v)
        pltpu.sync_copy(dest2_hbm.at[pl.ds(wid * n2, n2)], dest2_v)
        pltpu.sync_copy(idx1_hbm.at[pl.ds(wid * n1, n1)], idx1_v)
        pltpu.sync_copy(idx0_hbm.at[wid], idx0_v)

        base1 = wid * n1 * CH
        for b in range(n1):
            pltpu.make_async_copy(
                tab_hbm.at[idx1_v.at[b]], bufs1.at[b], gs_1).start()
        seed_cp = pltpu.async_copy(tab_hbm.at[idx0_v], rows_s, sem_s)

        gsem = (gs_a, gs_b)
        wsem = (ws_a, ws_b)

        def g_copy(c, b, ph):
            return pltpu.make_async_copy(
                tab_hbm.at[idx2_v.at[c]], bufs.at[ph * 4 + b], gsem[ph])

        def w_copy(c, b, ph):
            return pltpu.make_async_copy(
                bufs.at[ph * 4 + b], out2_hbm.at[dest2_v.at[c]], wsem[ph])

        def fire_g(g, ph):
            for b in range(4):
                g_copy(g * 4 + b, b, ph).start()

        def drain_g(g, ph):
            for b in range(4):
                g_copy(g * 4 + b, b, ph).wait()

        def fire_w(g, ph):
            for b in range(4):
                w_copy(g * 4 + b, b, ph).start()

        def drain_w(g, ph):
            for b in range(4):
                w_copy(g * 4 + b, b, ph).wait()

        fire_g(0, 0)
        fire_g(1, 1)

        def body(i, carry):
            del carry
            g0 = i * 2
            drain_g(g0, 0)
            fire_w(g0, 0)
            drain_g(g0 + 1, 1)
            fire_w(g0 + 1, 1)
            drain_w(g0, 0)
            fire_g(g0 + 2, 0)
            drain_w(g0 + 1, 1)
            fire_g(g0 + 3, 1)
            return 0

        lax.fori_loop(0, g2 // 2 - 1, body, 0)

        ge = g2 - 2
        drain_g(ge, 0)
        fire_w(ge, 0)
        drain_g(ge + 1, 1)
        fire_w(ge + 1, 1)
        drain_w(ge, 0)
        drain_w(ge + 1, 1)

        for b in range(n1):
            pltpu.make_async_copy(
                tab_hbm.at[idx1_v.at[b]], bufs1.at[b], gs_1).wait()
            pltpu.make_async_copy(
                bufs1.at[b], out1_hbm.at[pl.ds(base1 + b * CH, CH)],
                ws_1).start()
        seed_cp.wait()
        pltpu.sync_copy(rows_s, out0_hbm.at[pl.ds(wid * w0, w0)])
        for b in range(n1):
            pltpu.make_async_copy(
                bufs1.at[b], out1_hbm.at[pl.ds(base1 + b * CH, CH)],
                ws_1).wait()

    return k(table, idx2, idx1, idx0, dest2)


def _gat_block_f(hs, hn_list, wq, wkv, ws, sb2, ex2, dh):
    scale = float(HD) ** (-0.5)
    dn = (((1,), (1,)), ((), ()))
    dc = (((1,), (0,)), ((), ()))
    f32 = jnp.float32
    bf16 = jnp.bfloat16
    hs = hs.astype(bf16)
    q = lax.dot_general(hs, wq, dn, preferred_element_type=f32)
    qr = (q * scale).astype(bf16)
    nf = len(hn_list)
    vs = []
    ps = []
    for f, hn in enumerate(hn_list):
        hn = hn.astype(bf16)
        kv = lax.dot_general(hn, wkv, dc, preferred_element_type=f32)
        ps.append(kv[:, :HID].astype(bf16) * qr)
        vs.append(kv[:, HID:].astype(bf16))
    s_all = None
    for f in range(nf):
        s = lax.dot_general(ps[f], sb2[f], dc, preferred_element_type=f32)
        s_all = s if s_all is None else s_all + s
    s_all = jnp.clip(s_all, -75.0, 75.0)
    e = jnp.exp(s_all)
    eb = e.astype(bf16)
    den = lax.dot_general(eb, dh, dc, preferred_element_type=f32)
    agg = None
    for f in range(0, nf, 2):
        w2 = lax.dot_general(eb, ex2[f // 2], dc, preferred_element_type=f32)
        t = w2[:, :HID] * vs[f] + w2[:, HID:] * vs[f + 1]
        agg = t if agg is None else agg + t
    agg = agg / den
    return lax.dot_general(hs, ws, dn, preferred_element_type=f32) + agg


SEED_BLK = 64


def _fused_gat_kernel(h0_ref, h1s_ref, h2_ref, wq1_ref, wkv1_ref, ws1_ref,
                      wq2_ref, wkv2_ref, ws2_ref,
                      sb_ref, ex_ref, dh_ref, out_ref):
    n1 = FAN1 * SEED_BLK
    sb2 = [sb_ref[f] for f in range(FAN2)]
    ex2 = [ex_ref[f] for f in range(FAN2 // 2)]
    dh = dh_ref[...]
    h1s = h1s_ref[...].reshape(n1, EMB)
    hn1 = [h2_ref[f].reshape(n1, EMB) for f in range(FAN2)]
    h1 = _gat_block_f(h1s, hn1, wq1_ref[...], wkv1_ref[...], ws1_ref[...],
                      sb2, ex2, dh)
    h1 = jnp.maximum(h1, 0.0).astype(jnp.bfloat16)
    hn2 = [h1[j * SEED_BLK:(j + 1) * SEED_BLK] for j in range(FAN1)]
    out = _gat_block_f(h0_ref[...], hn2, wq2_ref[...], wkv2_ref[...],
                       ws2_ref[...], sb2, ex2, dh)
    out_ref[...] = jnp.maximum(out, 0.0)


def _tc_fused(h0, h1s, h2, wq1, wkv1, ws1, wq2, wkv2, ws2,
              sbig, exbig, dhm):
    nb = h0.shape[0]
    grid = nb // SEED_BLK
    wspec = pl.BlockSpec((HID, HID), lambda i: (0, 0))
    kvspec = pl.BlockSpec((HID, 2 * HID), lambda i: (0, 0))
    return pl.pallas_call(
        _fused_gat_kernel,
        grid=(grid,),
        in_specs=[
            pl.BlockSpec((SEED_BLK, EMB), lambda i: (i, 0)),
            pl.BlockSpec((FAN1, SEED_BLK, EMB), lambda i: (0, i, 0)),
            pl.BlockSpec((FAN2, FAN1, SEED_BLK, EMB), lambda i: (0, 0, i, 0)),
            wspec, kvspec, wspec, wspec, kvspec, wspec,
            pl.BlockSpec((FAN2, HID, HID), lambda i: (0, 0, 0)),
            pl.BlockSpec((FAN2 // 2, HID, 2 * HID), lambda i: (0, 0, 0)),
            wspec,
        ],
        out_specs=pl.BlockSpec((SEED_BLK, HID), lambda i: (i, 0)),
        out_shape=jax.ShapeDtypeStruct((nb, HID), jnp.float32),
    )(h0, h1s, h2, wq1, wkv1, ws1, wq2, wkv2, ws2, sbig, exbig, dhm)


def _indicators():
    f = jnp.arange(FAN2)[:, None, None]
    d = jnp.arange(HID)[None, :, None]
    c = jnp.arange(HID)[None, None, :]
    sbig = (c == 8 * f + d // HD).astype(jnp.bfloat16)
    exbig = jnp.swapaxes(sbig, 1, 2)
    exbig = jnp.concatenate(
        [exbig[0::2], exbig[1::2]], axis=2)
    cc = jnp.arange(HID)[:, None]
    dd = jnp.arange(HID)[None, :]
    dhm = (cc % HEADS == dd // HD).astype(jnp.bfloat16)
    return sbig, exbig, dhm


GROUP_SIZES = (256, 512, 512, 512, 256)


def kernel(seeds, nbr1, nbr2, emb, Wq1, Wk1, Wv1, Ws1, Wq2, Wk2, Wv2, Ws2):
    def dest_map(bg):
        p = jnp.arange(bg * FAN1 * FAN2, dtype=jnp.int32)
        return ((p % FAN2) * (FAN1 * bg) + ((p // FAN2) % FAN1) * bg
                + p // (FAN1 * FAN2)).reshape(-1, CH)

    dest2 = {bg: dest_map(bg) for bg in set(GROUP_SIZES)}
    nbr2_r = nbr2.reshape(-1).astype(jnp.int32)
    nbr1_t = nbr1.reshape(B, FAN1).T.astype(jnp.int32)
    seeds = seeds.astype(jnp.int32)
    bf = jnp.bfloat16
    wkv1 = jnp.concatenate([Wk1.T, Wv1.T], axis=1).astype(bf)
    wkv2 = jnp.concatenate([Wk2.T, Wv2.T], axis=1).astype(bf)
    sbig, exbig, dhm = _indicators()
    wq1, ws1 = Wq1.astype(bf), Ws1.astype(bf)
    wq2, ws2 = Wq2.astype(bf), Ws2.astype(bf)
    outs = []
    b0 = 0
    for bg in GROUP_SIZES:
        sl = slice(b0, b0 + bg)
        idx2 = nbr2_r[b0 * FAN1 * FAN2:(b0 + bg) * FAN1 * FAN2]
        idx2 = idx2.reshape(-1, CH)
        idx1 = nbr1_t[:, sl].reshape(-1, CH)
        idx0 = seeds[sl].reshape(NW, bg // NW)
        h2, h1s, h0 = _sc_gather_all(emb, idx2, idx1, idx0, dest2[bg])
        h2 = h2.reshape(FAN2, FAN1, bg, EMB)
        h1s = h1s.reshape(FAN1, bg, EMB)
        outs.append(_tc_fused(h0, h1s, h2, wq1, wkv1, ws1,
                              wq2, wkv2, ws2, sbig, exbig, dhm))
        b0 += bg
    return jnp.concatenate(outs, axis=0)
